# Initial kernel scaffold; baseline (speedup 1.0000x reference)
import jax
import jax.numpy as jnp
from jax.experimental import pallas as pl


def kernel(x, synset_table, pos_table, sense_table, lemma_table, ln_gamma, ln_beta):
    raise NotImplementedError("write your pallas kernel here")


# speedup vs baseline: 1.3326x; 1.3326x over previous
import functools

import jax
import jax.numpy as jnp
from jax import lax
from jax.experimental import pallas as pl
from jax.experimental.pallas import tpu as pltpu
from jax.experimental.pallas import tpu_sc as plsc

B = 16384
H = 128
L = 16
NC = 2
NS = 16
NW = NC * NS
CH = 128
CPW = B // (NW * CH)
NV = H // L
EPS = 1e-12


_GDN = lax.GatherDimensionNumbers(
    offset_dims=(), collapsed_slice_dims=(0,), start_index_map=(0,))


def _hsum(v):
    for sh in (8, 4, 2, 1):
        perm = lax.iota(jnp.int32, L) ^ sh
        v = v + lax.gather(v, perm[:, None], _GDN, slice_sizes=(1,),
                           mode=lax.GatherScatterMode.PROMISE_IN_BOUNDS)
    return v


def _rsqrt_vec(v):
    i = lax.bitcast_convert_type(v, jnp.int32)
    i = jnp.int32(0x5F3759DF) - (i >> 1)
    y = lax.bitcast_convert_type(i, jnp.float32)
    for _ in range(3):
        y = y * (1.5 - 0.5 * v * y * y)
    return y


def _sc_body(xT, syn, pos, sen, lem, gam, bet, out,
             idx_v, r0, r1, r2, r3, ob, g_v, b_v, sem):
    wid = lax.axis_index("s") * NC + lax.axis_index("c")
    cbase = wid * CPW
    for t in range(4):
        pltpu.sync_copy(xT.at[t, pl.ds(cbase, CPW)], idx_v.at[t])
    pltpu.sync_copy(gam, g_v)
    pltpu.sync_copy(bet, b_v)
    tables = (syn, pos, sen, lem)
    bufs = (r0, r1, r2, r3)
    for c in range(CPW):
        cps = [pltpu.async_copy(tables[t].at[idx_v.at[t, c]], bufs[t], sem)
               for t in range(4)]
        for cp in cps:
            cp.wait()

        def row(r, carry):
            es = []
            s = None
            q = None
            for j in range(NV):
                v = (r0[r, pl.ds(j * L, L)] + r1[r, pl.ds(j * L, L)]
                     + r2[r, pl.ds(j * L, L)] + r3[r, pl.ds(j * L, L)])
                es.append(v)
                s = v if s is None else s + v
                q = v * v if q is None else q + v * v
            mean = _hsum(s) * (1.0 / H)
            msq = _hsum(q) * (1.0 / H)
            var = msq - mean * mean
            rstd = _rsqrt_vec(var + EPS)
            for j in range(NV):
                gj = g_v[pl.ds(j * L, L)]
                bj = b_v[pl.ds(j * L, L)]
                ob[r, pl.ds(j * L, L)] = (es[j] - mean) * rstd * gj + bj
            return carry

        lax.fori_loop(0, CH, row, 0)
        pltpu.sync_copy(ob, out.at[pl.ds((cbase + c) * CH, CH)])


_mesh = plsc.VectorSubcoreMesh(core_axis_name="c", subcore_axis_name="s")

_embed_ln = functools.partial(
    pl.kernel,
    out_type=jax.ShapeDtypeStruct((B, H), jnp.float32),
    mesh=_mesh,
    scratch_types=[
        pltpu.VMEM((4, CPW, CH), jnp.int32),
        pltpu.VMEM((CH, H), jnp.float32),
        pltpu.VMEM((CH, H), jnp.float32),
        pltpu.VMEM((CH, H), jnp.float32),
        pltpu.VMEM((CH, H), jnp.float32),
        pltpu.VMEM((CH, H), jnp.float32),
        pltpu.VMEM((H,), jnp.float32),
        pltpu.VMEM((H,), jnp.float32),
        pltpu.SemaphoreType.DMA,
    ],
)(_sc_body)


@jax.jit
def kernel(x, synset_table, pos_table, sense_table, lemma_table,
           ln_gamma, ln_beta):
    xT = jnp.asarray(x, jnp.int32).T.reshape(4, B // CH, CH)
    return _embed_ln(xT, synset_table, pos_table, sense_table, lemma_table,
                     ln_gamma, ln_beta)


# speedup vs baseline: 1.4711x; 1.1039x over previous
import functools

import jax
import jax.numpy as jnp
from jax import lax
from jax.experimental import pallas as pl
from jax.experimental.pallas import tpu as pltpu
from jax.experimental.pallas import tpu_sc as plsc

B = 16384
H = 128
L = 16
NC = 2
NS = 16
NW = NC * NS
CH = 128
CPW = B // (NW * CH)
NV = H // L
EPS = 1e-12

_GDN = lax.GatherDimensionNumbers(
    offset_dims=(), collapsed_slice_dims=(0,), start_index_map=(0,))


def _hsum(v):
    for sh in (8, 4, 2, 1):
        perm = lax.iota(jnp.int32, L) ^ sh
        v = v + lax.gather(v, perm[:, None], _GDN, slice_sizes=(1,),
                           mode=lax.GatherScatterMode.PROMISE_IN_BOUNDS)
    return v


def _rsqrt_vec(v):
    i = lax.bitcast_convert_type(v, jnp.int32)
    i = jnp.int32(0x5F3759DF) - (i >> 1)
    y = lax.bitcast_convert_type(i, jnp.float32)
    for _ in range(3):
        y = y * (1.5 - 0.5 * v * y * y)
    return y


def _sc_body(xT, syn, pos, sen, lem, gam, bet, out,
             idx_v, bufA, bufB, outA, outB, g_v, b_v, semA, semB, semO):
    wid = lax.axis_index("s") * NC + lax.axis_index("c")
    cbase = wid * CPW
    for t in range(4):
        pltpu.sync_copy(xT.at[t, pl.ds(cbase, CPW)], idx_v.at[t])
    pltpu.sync_copy(gam, g_v)
    pltpu.sync_copy(bet, b_v)

    tables = (syn, pos, sen, lem)
    bufs = (bufA, bufB)
    outs = (outA, outB)
    sems = (semA, semB)
    zero = jnp.zeros((L,), jnp.float32)

    def zrow(r, carry, *, buf):
        for j in range(NV):
            buf[r, pl.ds(j * L, L)] = zero
        return carry

    lax.fori_loop(0, CH, functools.partial(zrow, buf=bufA), 0)
    lax.fori_loop(0, CH, functools.partial(zrow, buf=bufB), 0)

    def gathers(c):
        buf, sem = bufs[c % 2], sems[c % 2]
        return [pltpu.async_copy(tables[t].at[idx_v.at[t, c]], buf, sem,
                                 add=True)
                for t in range(4)]

    pend = {0: gathers(0), 1: gathers(1)}
    ostores = {}
    for c in range(CPW):
        buf, ob = bufs[c % 2], outs[c % 2]
        if c - 2 in ostores:
            ostores.pop(c - 2).wait()
        for cp in pend.pop(c):
            cp.wait()

        def row(r, carry):
            es = []
            s = None
            for j in range(NV):
                v = buf[r, pl.ds(j * L, L)]
                buf[r, pl.ds(j * L, L)] = zero
                es.append(v)
                s = v if s is None else s + v
            q = None
            for j in range(NV):
                p = es[j] * es[j]
                q = p if q is None else q + p
            mean = _hsum(s) * (1.0 / H)
            msq = _hsum(q) * (1.0 / H)
            rstd = _rsqrt_vec(msq - mean * mean + EPS)
            for j in range(NV):
                gj = g_v[pl.ds(j * L, L)]
                bj = b_v[pl.ds(j * L, L)]
                ob[r, pl.ds(j * L, L)] = (es[j] - mean) * rstd * gj + bj
            return carry

        lax.fori_loop(0, CH, row, 0)
        if c + 2 < CPW:
            pend[c + 2] = gathers(c + 2)
        ostores[c] = pltpu.async_copy(
            ob, out.at[pl.ds((cbase + c) * CH, CH)], semO)
    for cp in ostores.values():
        cp.wait()


_mesh = plsc.VectorSubcoreMesh(core_axis_name="c", subcore_axis_name="s")

_embed_ln = functools.partial(
    pl.kernel,
    out_type=jax.ShapeDtypeStruct((B, H), jnp.float32),
    mesh=_mesh,
    scratch_types=[
        pltpu.VMEM((4, CPW, CH), jnp.int32),
        pltpu.VMEM((CH, H), jnp.float32),
        pltpu.VMEM((CH, H), jnp.float32),
        pltpu.VMEM((CH, H), jnp.float32),
        pltpu.VMEM((CH, H), jnp.float32),
        pltpu.VMEM((H,), jnp.float32),
        pltpu.VMEM((H,), jnp.float32),
        pltpu.SemaphoreType.DMA,
        pltpu.SemaphoreType.DMA,
        pltpu.SemaphoreType.DMA,
    ],
)(_sc_body)


@jax.jit
def kernel(x, synset_table, pos_table, sense_table, lemma_table,
           ln_gamma, ln_beta):
    xT = jnp.asarray(x, jnp.int32).T.reshape(4, B // CH, CH)
    return _embed_ln(xT, synset_table, pos_table, sense_table, lemma_table,
                     ln_gamma, ln_beta)


# speedup vs baseline: 1.6371x; 1.1128x over previous
import functools

import jax
import jax.numpy as jnp
from jax import lax
from jax.experimental import pallas as pl
from jax.experimental.pallas import tpu as pltpu
from jax.experimental.pallas import tpu_sc as plsc

B = 16384
H = 128
L = 16
NC = 2
NS = 16
NW = NC * NS
CH = 128
CPW = B // (NW * CH)
NV = H // L
EPS = 1e-12

_GDN = lax.GatherDimensionNumbers(
    offset_dims=(), collapsed_slice_dims=(0,), start_index_map=(0,))


def _hsum(v):
    for sh in (8, 4, 2, 1):
        perm = lax.iota(jnp.int32, L) ^ sh
        v = v + lax.gather(v, perm[:, None], _GDN, slice_sizes=(1,),
                           mode=lax.GatherScatterMode.PROMISE_IN_BOUNDS)
    return v


def _rsqrt_vec(v):
    i = lax.bitcast_convert_type(v, jnp.int32)
    i = jnp.int32(0x5F3759DF) - (i >> 1)
    y = lax.bitcast_convert_type(i, jnp.float32)
    for _ in range(2):
        y = y * (1.5 - 0.5 * v * y * y)
    return y


def _sc_body(xT, syn, pos, sen, lem, gam, bet, out,
             idx_v, bufA, bufB, outA, outB, g_v, b_v, semA, semB, semO):
    wid = lax.axis_index("s") * NC + lax.axis_index("c")
    cbase = wid * CPW
    for t in range(4):
        pltpu.sync_copy(xT.at[t, pl.ds(cbase, CPW)], idx_v.at[t])
    pltpu.sync_copy(gam, g_v)
    pltpu.sync_copy(bet, b_v)

    tables = (syn, pos, sen, lem)
    bufs = (bufA, bufB)
    outs = (outA, outB)
    sems = (semA, semB)
    zero = jnp.zeros((L,), jnp.float32)

    def zrow(r, carry, *, buf):
        for j in range(NV):
            buf[r, pl.ds(j * L, L)] = zero
        return carry

    lax.fori_loop(0, CH, functools.partial(zrow, buf=bufA), 0)
    lax.fori_loop(0, CH, functools.partial(zrow, buf=bufB), 0)

    def gathers(c):
        buf, sem = bufs[c % 2], sems[c % 2]
        return [pltpu.async_copy(tables[t].at[idx_v.at[t, c]], buf, sem,
                                 add=True)
                for t in range(4)]

    pend = {0: gathers(0), 1: gathers(1)}
    ostores = {}
    for c in range(CPW):
        buf, ob = bufs[c % 2], outs[c % 2]
        if c - 2 in ostores:
            ostores.pop(c - 2).wait()
        for cp in pend.pop(c):
            cp.wait()

        def row2(i, gb):
            for r in (i * 2, i * 2 + 1):
                es = []
                s = None
                for j in range(NV):
                    v = buf[r, pl.ds(j * L, L)]
                    buf[r, pl.ds(j * L, L)] = zero
                    es.append(v)
                    s = v if s is None else s + v
                q = None
                for j in range(NV):
                    p = es[j] * es[j]
                    q = p if q is None else q + p
                mean = _hsum(s) * (1.0 / H)
                msq = _hsum(q) * (1.0 / H)
                rstd = _rsqrt_vec(msq - mean * mean + EPS)
                for j in range(NV):
                    ob[r, pl.ds(j * L, L)] = \
                        (es[j] - mean) * rstd * gb[j] + gb[NV + j]
            return gb

        gb0 = tuple(g_v[pl.ds(j * L, L)] for j in range(NV)) + \
            tuple(b_v[pl.ds(j * L, L)] for j in range(NV))
        lax.fori_loop(0, CH // 2, row2, gb0)
        if c + 2 < CPW:
            pend[c + 2] = gathers(c + 2)
        ostores[c] = pltpu.async_copy(
            ob, out.at[pl.ds((cbase + c) * CH, CH)], semO)
    for cp in ostores.values():
        cp.wait()


_mesh = plsc.VectorSubcoreMesh(core_axis_name="c", subcore_axis_name="s")

_embed_ln = functools.partial(
    pl.kernel,
    out_type=jax.ShapeDtypeStruct((B, H), jnp.float32),
    mesh=_mesh,
    scratch_types=[
        pltpu.VMEM((4, CPW, CH), jnp.int32),
        pltpu.VMEM((CH, H), jnp.float32),
        pltpu.VMEM((CH, H), jnp.float32),
        pltpu.VMEM((CH, H), jnp.float32),
        pltpu.VMEM((CH, H), jnp.float32),
        pltpu.VMEM((H,), jnp.float32),
        pltpu.VMEM((H,), jnp.float32),
        pltpu.SemaphoreType.DMA,
        pltpu.SemaphoreType.DMA,
        pltpu.SemaphoreType.DMA,
    ],
)(_sc_body)


@jax.jit
def kernel(x, synset_table, pos_table, sense_table, lemma_table,
           ln_gamma, ln_beta):
    xT = jnp.asarray(x, jnp.int32).T.reshape(4, B // CH, CH)
    return _embed_ln(xT, synset_table, pos_table, sense_table, lemma_table,
                     ln_gamma, ln_beta)
